# Initial kernel scaffold; baseline (speedup 1.0000x reference)
import jax
import jax.numpy as jnp
from jax.experimental import pallas as pl


def kernel(x, W_slr, b_slr, W_out, b_out):
    raise NotImplementedError("write your pallas kernel here")


# speedup vs baseline: 2.1982x; 2.1982x over previous
import jax
import jax.numpy as jnp
from jax import lax
from jax.experimental import pallas as pl

N_S = 4
N_LR = 22
K = 40
V = 512


def _gravnet_body(x_ref, wslrT_ref, bslr_ref, woutT_ref, bo_ref, o_ref):
    x = x_ref[0]
    means = jnp.mean(x, axis=0, keepdims=True)
    x2 = jnp.concatenate([x, jnp.broadcast_to(means, x.shape)], axis=1)

    slr = jax.nn.relu(jnp.dot(x2, wslrT_ref[...],
                              preferred_element_type=jnp.float32)
                      + bslr_ref[...])
    s = slr[:, :N_S]
    lr = slr[:, N_S:N_S + N_LR]
    sT = slr[:, :N_S].T
    lrT = lr.T

    sq = jnp.zeros((V, V), jnp.float32)
    for c in range(N_S):
        diff = s[:, c:c + 1] - sT[c:c + 1, :]
        sq = sq + diff * diff
    d = jnp.where(sq > 0.0, jnp.sqrt(jnp.where(sq > 0.0, sq, 1.0)), 0.0)

    jidx = lax.broadcasted_iota(jnp.int32, (V, V), 1)

    def step(_, carry):
        dcur, wm = carry
        m = jnp.min(dcur, axis=1, keepdims=True)
        is_min = dcur == m
        fj = jnp.min(jnp.where(is_min, jidx, V), axis=1, keepdims=True)
        one_hot = is_min & (jidx == fj)
        w = jnp.exp(-10.0 * m * m)
        wm = wm + jnp.where(one_hot, w, 0.0)
        dcur = jnp.where(one_hot, jnp.float32(jnp.inf), dcur)
        return dcur, wm

    _, wm = lax.fori_loop(0, K, step, (d, jnp.zeros((V, V), jnp.float32)))

    mean_agg = lax.dot_general(wm, lr, (((1,), (0,)), ((), ())),
                               preferred_element_type=jnp.float32,
                               precision=lax.Precision.HIGHEST) * (1.0 / K)

    cols = [jnp.max(wm * lrT[f:f + 1, :], axis=1, keepdims=True)
            for f in range(N_LR)]
    max_agg = jnp.concatenate(cols, axis=1)

    fp = jnp.concatenate([x2, mean_agg, max_agg], axis=1)
    out = jnp.dot(fp, woutT_ref[...],
                  preferred_element_type=jnp.float32) + bo_ref[...]
    o_ref[0] = jax.nn.relu(out)


def kernel(x, W_slr, b_slr, W_out, b_out):
    B, V_, F = x.shape
    wslrT = W_slr.T
    bslr = b_slr.reshape(1, -1)
    woutT = W_out.T
    bo = b_out.reshape(1, -1)
    n_out = W_out.shape[0]

    full = lambda shape: pl.BlockSpec(shape, lambda b: (0,) * len(shape))
    return pl.pallas_call(
        _gravnet_body,
        grid=(B,),
        in_specs=[
            pl.BlockSpec((1, V_, F), lambda b: (b, 0, 0)),
            full(wslrT.shape), full(bslr.shape),
            full(woutT.shape), full(bo.shape),
        ],
        out_specs=pl.BlockSpec((1, V_, n_out), lambda b: (b, 0, 0)),
        out_shape=jax.ShapeDtypeStruct((B, V_, n_out), jnp.float32),
    )(x, wslrT, bslr, woutT, bo)


# speedup vs baseline: 5.0479x; 2.2963x over previous
import jax
import jax.numpy as jnp
from jax import lax
from jax.experimental import pallas as pl

N_S = 4
N_LR = 22
K = 40
V = 512


def _gravnet_body(x_ref, wslrT_ref, bslr_ref, woutT_ref, bo_ref, o_ref):
    x = x_ref[0]
    means = jnp.mean(x, axis=0, keepdims=True)
    x2 = jnp.concatenate([x, jnp.broadcast_to(means, x.shape)], axis=1)

    slr = jax.nn.relu(jnp.dot(x2, wslrT_ref[...],
                              preferred_element_type=jnp.float32)
                      + bslr_ref[...])
    s = slr[:, :N_S]
    lr = slr[:, N_S:N_S + N_LR]
    sT = slr[:, :N_S].T
    lrT = lr.T

    sq = jnp.zeros((V, V), jnp.float32)
    for c in range(N_S):
        diff = s[:, c:c + 1] - sT[c:c + 1, :]
        sq = sq + diff * diff
    d = jnp.where(sq > 0.0, jnp.sqrt(jnp.where(sq > 0.0, sq, 1.0)), 0.0)

    dbits = lax.bitcast_convert_type(d, jnp.int32)

    def bstep(_, carry):
        lo, hi = carry
        mid = lo + lax.shift_right_logical(hi - lo, 1)
        cnt = jnp.sum((dbits <= mid).astype(jnp.int32), axis=1, keepdims=True)
        pred = cnt >= K
        return jnp.where(pred, lo, mid + 1), jnp.where(pred, mid, hi)

    lo0 = jnp.zeros((V, 1), jnp.int32)
    hi0 = jnp.full((V, 1), 0x7f800000, jnp.int32)
    t, _ = lax.fori_loop(0, 31, bstep, (lo0, hi0))

    lt = dbits < t
    tie = dbits == t
    need = (K - jnp.sum(lt.astype(jnp.int32), axis=1, keepdims=True)
            ).astype(jnp.float32)
    jr = lax.broadcasted_iota(jnp.int32, (V, V), 0)
    jc = lax.broadcasted_iota(jnp.int32, (V, V), 1)
    ltri = (jr < jc).astype(jnp.float32)
    tie_rank = lax.dot_general(tie.astype(jnp.float32), ltri,
                               (((1,), (0,)), ((), ())),
                               preferred_element_type=jnp.float32)
    sel = lt | (tie & (tie_rank < need))
    wm = jnp.where(sel, jnp.exp(-10.0 * d * d), 0.0)

    mean_agg = lax.dot_general(wm, lr, (((1,), (0,)), ((), ())),
                               preferred_element_type=jnp.float32,
                               precision=lax.Precision.HIGHEST) * (1.0 / K)

    cols = [jnp.max(wm * lrT[f:f + 1, :], axis=1, keepdims=True)
            for f in range(N_LR)]
    max_agg = jnp.concatenate(cols, axis=1)

    fp = jnp.concatenate([x2, mean_agg, max_agg], axis=1)
    out = jnp.dot(fp, woutT_ref[...],
                  preferred_element_type=jnp.float32) + bo_ref[...]
    o_ref[0] = jax.nn.relu(out)


def kernel(x, W_slr, b_slr, W_out, b_out):
    B, V_, F = x.shape
    wslrT = W_slr.T
    bslr = b_slr.reshape(1, -1)
    woutT = W_out.T
    bo = b_out.reshape(1, -1)
    n_out = W_out.shape[0]

    full = lambda shape: pl.BlockSpec(shape, lambda b: (0,) * len(shape))
    return pl.pallas_call(
        _gravnet_body,
        grid=(B,),
        in_specs=[
            pl.BlockSpec((1, V_, F), lambda b: (b, 0, 0)),
            full(wslrT.shape), full(bslr.shape),
            full(woutT.shape), full(bo.shape),
        ],
        out_specs=pl.BlockSpec((1, V_, n_out), lambda b: (b, 0, 0)),
        out_shape=jax.ShapeDtypeStruct((B, V_, n_out), jnp.float32),
    )(x, wslrT, bslr, woutT, bo)


# speedup vs baseline: 8.6022x; 1.7041x over previous
import jax
import jax.numpy as jnp
from jax import lax
from jax.experimental import pallas as pl

N_S = 4
N_LR = 22
K = 40
V = 512


def _gravnet_body(x_ref, wslrT_ref, bslr_ref, woutT_ref, bo_ref, o_ref):
    x = x_ref[0]
    means = jnp.mean(x, axis=0, keepdims=True)
    x2 = jnp.concatenate([x, jnp.broadcast_to(means, x.shape)], axis=1)

    slr = jax.nn.relu(jnp.dot(x2, wslrT_ref[...],
                              preferred_element_type=jnp.float32)
                      + bslr_ref[...])
    s = slr[:, :N_S]
    lr = slr[:, N_S:N_S + N_LR]
    sT = slr[:, :N_S].T

    sq = jnp.zeros((V, V), jnp.float32)
    for c in range(N_S):
        diff = s[:, c:c + 1] - sT[c:c + 1, :]
        sq = sq + diff * diff
    d = jnp.where(sq > 0.0, jnp.sqrt(jnp.where(sq > 0.0, sq, 1.0)), 0.0)

    dbits = lax.bitcast_convert_type(d, jnp.int32)

    def bstep(_, carry):
        lo, hi = carry
        mid = lo + lax.shift_right_logical(hi - lo, 1)
        cnt = jnp.sum((dbits <= mid).astype(jnp.int32), axis=0, keepdims=True)
        pred = cnt >= K
        return jnp.where(pred, lo, mid + 1), jnp.where(pred, mid, hi)

    t, _ = lax.fori_loop(0, 31, bstep,
                         (jnp.zeros((1, V), jnp.int32),
                          jnp.full((1, V), 0x7f800000, jnp.int32)))

    ltc = dbits < t
    tiec = dbits == t
    need = (K - jnp.sum(ltc.astype(jnp.int32), axis=0, keepdims=True)
            ).astype(jnp.float32)
    ir = lax.broadcasted_iota(jnp.int32, (V, V), 0)
    ic = lax.broadcasted_iota(jnp.int32, (V, V), 1)
    ltri = (ir > ic).astype(jnp.float32)
    tie_rank = lax.dot_general(ltri, tiec.astype(jnp.float32),
                               (((1,), (0,)), ((), ())),
                               preferred_element_type=jnp.float32)
    selc = ltc | (tiec & (tie_rank < need))
    wmT = jnp.where(selc, jnp.exp(-10.0 * d * d), 0.0)

    mean_agg = lax.dot_general(wmT, lr, (((0,), (0,)), ((), ())),
                               preferred_element_type=jnp.float32,
                               precision=lax.Precision.HIGHEST) * (1.0 / K)

    rows = [jnp.max(wmT * lr[:, f:f + 1], axis=0, keepdims=True)
            for f in range(N_LR)]
    max_agg = jnp.concatenate(rows, axis=0).T

    fp = jnp.concatenate([x2, mean_agg, max_agg], axis=1)
    out = jnp.dot(fp, woutT_ref[...],
                  preferred_element_type=jnp.float32) + bo_ref[...]
    o_ref[0] = jax.nn.relu(out)


def kernel(x, W_slr, b_slr, W_out, b_out):
    B, V_, F = x.shape
    wslrT = W_slr.T
    bslr = b_slr.reshape(1, -1)
    woutT = W_out.T
    bo = b_out.reshape(1, -1)
    n_out = W_out.shape[0]

    full = lambda shape: pl.BlockSpec(shape, lambda b: (0,) * len(shape))
    return pl.pallas_call(
        _gravnet_body,
        grid=(B,),
        in_specs=[
            pl.BlockSpec((1, V_, F), lambda b: (b, 0, 0)),
            full(wslrT.shape), full(bslr.shape),
            full(woutT.shape), full(bo.shape),
        ],
        out_specs=pl.BlockSpec((1, V_, n_out), lambda b: (b, 0, 0)),
        out_shape=jax.ShapeDtypeStruct((B, V_, n_out), jnp.float32),
    )(x, wslrT, bslr, woutT, bo)


# speedup vs baseline: 8.9637x; 1.0420x over previous
import jax
import jax.numpy as jnp
from jax import lax
from jax.experimental import pallas as pl

N_S = 4
N_LR = 22
K = 40
V = 512


def _gravnet_body(x_ref, wslrT_ref, bslr_ref, woutT_ref, bo_ref, o_ref):
    B = x_ref.shape[0]
    x2s = []
    for b in range(B):
        xb = x_ref[b]
        means = jnp.mean(xb, axis=0, keepdims=True)
        x2s.append(jnp.concatenate(
            [xb, jnp.broadcast_to(means, xb.shape)], axis=1))
    x2 = jnp.concatenate(x2s, axis=0)

    slr = jax.nn.relu(jnp.dot(x2, wslrT_ref[...],
                              preferred_element_type=jnp.float32)
                      + bslr_ref[...])
    s = slr[:, :N_S]
    lr = slr[:, N_S:N_S + N_LR]

    ones_row = jnp.ones((1, N_S), jnp.float32)
    sq_blocks = []
    for b in range(B):
        sb = s[b * V:(b + 1) * V]
        u = sb * sb
        ssq_col = jnp.sum(u, axis=1, keepdims=True)
        ssq_row = lax.dot_general(ones_row, u, (((1,), (1,)), ((), ())),
                                  preferred_element_type=jnp.float32,
                                  precision=lax.Precision.HIGHEST)
        gram = lax.dot_general(sb, sb, (((1,), (1,)), ((), ())),
                               preferred_element_type=jnp.float32,
                               precision=lax.Precision.HIGHEST)
        sq_blocks.append(jnp.maximum(ssq_col + ssq_row - 2.0 * gram, 0.0))
    sq = jnp.concatenate(sq_blocks, axis=1)

    dbits = lax.bitcast_convert_type(sq, jnp.int32)
    BV = B * V

    def bstep(_, carry):
        lo, hi = carry
        mid = lo + lax.shift_right_logical(hi - lo, 1)
        cnt = jnp.sum((dbits <= mid).astype(jnp.int32), axis=0, keepdims=True)
        pred = cnt >= K
        return jnp.where(pred, lo, mid + 1), jnp.where(pred, mid, hi)

    t, _ = lax.fori_loop(0, 31, bstep,
                         (jnp.zeros((1, BV), jnp.int32),
                          jnp.full((1, BV), 0x7f800000, jnp.int32)))

    ltc = dbits < t
    tiec = dbits == t
    need = (K - jnp.sum(ltc.astype(jnp.int32), axis=0, keepdims=True)
            ).astype(jnp.float32)
    ir = lax.broadcasted_iota(jnp.int32, (V, V), 0)
    ic = lax.broadcasted_iota(jnp.int32, (V, V), 1)
    ltri = (ir > ic).astype(jnp.float32)
    tie_rank = lax.dot_general(ltri, tiec.astype(jnp.float32),
                               (((1,), (0,)), ((), ())),
                               preferred_element_type=jnp.float32)
    selc = ltc | (tiec & (tie_rank < need))
    wmT = jnp.where(selc, jnp.exp(-10.0 * sq), 0.0)

    mean_blocks = []
    max_blocks = []
    for b in range(B):
        wb = wmT[:, b * V:(b + 1) * V]
        lrb = lr[b * V:(b + 1) * V]
        mean_blocks.append(
            lax.dot_general(wb, lrb, (((0,), (0,)), ((), ())),
                            preferred_element_type=jnp.float32,
                            precision=lax.Precision.HIGHEST) * (1.0 / K))
        rows = [jnp.max(wb * lrb[:, f:f + 1], axis=0, keepdims=True)
                for f in range(N_LR)]
        max_blocks.append(jnp.concatenate(rows, axis=0).T)
    mean_agg = jnp.concatenate(mean_blocks, axis=0)
    max_agg = jnp.concatenate(max_blocks, axis=0)

    fp = jnp.concatenate([x2, mean_agg, max_agg], axis=1)
    out = jnp.dot(fp, woutT_ref[...],
                  preferred_element_type=jnp.float32) + bo_ref[...]
    o_ref[...] = jax.nn.relu(out).reshape(B, V, -1)


def kernel(x, W_slr, b_slr, W_out, b_out):
    B, V_, F = x.shape
    wslrT = W_slr.T
    bslr = b_slr.reshape(1, -1)
    woutT = W_out.T
    bo = b_out.reshape(1, -1)
    n_out = W_out.shape[0]

    full = lambda shape: pl.BlockSpec(shape, lambda: (0,) * len(shape))
    return pl.pallas_call(
        _gravnet_body,
        in_specs=[
            full(x.shape),
            full(wslrT.shape), full(bslr.shape),
            full(woutT.shape), full(bo.shape),
        ],
        out_specs=full((B, V_, n_out)),
        out_shape=jax.ShapeDtypeStruct((B, V_, n_out), jnp.float32),
    )(x, wslrT, bslr, woutT, bo)


# speedup vs baseline: 9.6719x; 1.0790x over previous
import jax
import jax.numpy as jnp
from jax import lax
from jax.experimental import pallas as pl

N_S = 4
N_LR = 22
K = 40
V = 512


def _gravnet_body(x_ref, wslrT_ref, bslr_ref, woutT_ref, bo_ref, o_ref):
    B = x_ref.shape[0]
    x2s = []
    for b in range(B):
        xb = x_ref[b]
        means = jnp.mean(xb, axis=0, keepdims=True)
        x2s.append(jnp.concatenate(
            [xb, jnp.broadcast_to(means, xb.shape)], axis=1))
    x2 = jnp.concatenate(x2s, axis=0)

    slr = jax.nn.relu(jnp.dot(x2, wslrT_ref[...],
                              preferred_element_type=jnp.float32)
                      + bslr_ref[...])
    s = slr[:, :N_S]
    lr = slr[:, N_S:N_S + N_LR]

    ones_row = jnp.ones((1, N_S), jnp.float32)
    sq_blocks = []
    for b in range(B):
        sb = s[b * V:(b + 1) * V]
        u = sb * sb
        ssq_col = jnp.sum(u, axis=1, keepdims=True)
        ssq_row = lax.dot_general(ones_row, u, (((1,), (1,)), ((), ())),
                                  preferred_element_type=jnp.float32,
                                  precision=lax.Precision.HIGHEST)
        gram = lax.dot_general(sb, sb, (((1,), (1,)), ((), ())),
                               preferred_element_type=jnp.float32,
                               precision=lax.Precision.HIGHEST)
        sq_blocks.append(jnp.maximum(ssq_col + ssq_row - 2.0 * gram, 0.0))
    sq = jnp.concatenate(sq_blocks, axis=1)

    dbits = lax.bitcast_convert_type(sq, jnp.int32)
    BV = B * V

    def bstep(_, carry):
        lo, hi = carry
        mid = lo + lax.shift_right_logical(hi - lo, 1)
        cnt = jnp.sum((dbits <= mid).astype(jnp.int32), axis=0, keepdims=True)
        pred = cnt >= K
        return jnp.where(pred, lo, mid + 1), jnp.where(pred, mid, hi)

    t, _ = lax.fori_loop(0, 31, bstep,
                         (jnp.zeros((1, BV), jnp.int32),
                          jnp.full((1, BV), 0x7f800000, jnp.int32)))

    ltc = dbits < t
    tiec = dbits == t
    need = (K - jnp.sum(ltc.astype(jnp.int32), axis=0, keepdims=True)
            ).astype(jnp.float32)
    ir = lax.broadcasted_iota(jnp.int32, (V, V), 0)
    ic = lax.broadcasted_iota(jnp.int32, (V, V), 1)
    ltri = (ir > ic).astype(jnp.float32)
    tie_rank = lax.dot_general(ltri, tiec.astype(jnp.float32),
                               (((1,), (0,)), ((), ())),
                               preferred_element_type=jnp.float32)
    selc = ltc | (tiec & (tie_rank < need))
    wmT = jnp.where(selc, jnp.exp(-10.0 * sq), 0.0)

    wmT16 = wmT.astype(jnp.bfloat16)
    lr16 = lr.astype(jnp.bfloat16)
    mean_blocks = []
    max_blocks = []
    for b in range(B):
        wb = wmT[:, b * V:(b + 1) * V]
        lrb = lr[b * V:(b + 1) * V]
        mean_blocks.append(
            lax.dot_general(wb, lrb, (((0,), (0,)), ((), ())),
                            preferred_element_type=jnp.float32,
                            precision=lax.Precision.HIGHEST) * (1.0 / K))
        wb16 = wmT16[:, b * V:(b + 1) * V]
        lrb16 = lr16[b * V:(b + 1) * V]
        rows = [jnp.max(wb16 * lrb16[:, f:f + 1], axis=0, keepdims=True)
                for f in range(N_LR)]
        max_blocks.append(
            jnp.concatenate(rows, axis=0).T.astype(jnp.float32))
    mean_agg = jnp.concatenate(mean_blocks, axis=0)
    max_agg = jnp.concatenate(max_blocks, axis=0)

    fp = jnp.concatenate([x2, mean_agg, max_agg], axis=1)
    out = jnp.dot(fp, woutT_ref[...],
                  preferred_element_type=jnp.float32) + bo_ref[...]
    o_ref[...] = jax.nn.relu(out).reshape(B, V, -1)


def kernel(x, W_slr, b_slr, W_out, b_out):
    B, V_, F = x.shape
    wslrT = W_slr.T
    bslr = b_slr.reshape(1, -1)
    woutT = W_out.T
    bo = b_out.reshape(1, -1)
    n_out = W_out.shape[0]

    full = lambda shape: pl.BlockSpec(shape, lambda: (0,) * len(shape))
    return pl.pallas_call(
        _gravnet_body,
        in_specs=[
            full(x.shape),
            full(wslrT.shape), full(bslr.shape),
            full(woutT.shape), full(bo.shape),
        ],
        out_specs=full((B, V_, n_out)),
        out_shape=jax.ShapeDtypeStruct((B, V_, n_out), jnp.float32),
    )(x, wslrT, bslr, woutT, bo)


# speedup vs baseline: 10.0114x; 1.0351x over previous
import jax
import jax.numpy as jnp
from jax import lax
from jax.experimental import pallas as pl

N_S = 4
N_LR = 22
K = 40
V = 512


def _gravnet_body(x_ref, wslrT_ref, bslr_ref, woutT_ref, bo_ref, o_ref):
    B = x_ref.shape[0]
    x2s = []
    for b in range(B):
        xb = x_ref[b]
        means = jnp.mean(xb, axis=0, keepdims=True)
        x2s.append(jnp.concatenate(
            [xb, jnp.broadcast_to(means, xb.shape)], axis=1))
    x2 = jnp.concatenate(x2s, axis=0)

    slr = jax.nn.relu(jnp.dot(x2, wslrT_ref[...],
                              preferred_element_type=jnp.float32)
                      + bslr_ref[...])
    s = slr[:, :N_S]
    lr = slr[:, N_S:N_S + N_LR]

    ones_row = jnp.ones((1, N_S), jnp.float32)
    sq_blocks = []
    for b in range(B):
        sb = s[b * V:(b + 1) * V]
        u = sb * sb
        ssq_col = jnp.sum(u, axis=1, keepdims=True)
        ssq_row = lax.dot_general(ones_row, u, (((1,), (1,)), ((), ())),
                                  preferred_element_type=jnp.float32,
                                  precision=lax.Precision.HIGHEST)
        gram = lax.dot_general(sb, sb, (((1,), (1,)), ((), ())),
                               preferred_element_type=jnp.float32,
                               precision=lax.Precision.HIGHEST)
        sq_blocks.append(jnp.maximum(ssq_col + ssq_row - 2.0 * gram, 0.0))
    sq = jnp.concatenate(sq_blocks, axis=1)

    dbits = lax.bitcast_convert_type(sq, jnp.int32)
    BV = B * V

    ones_cnt = jnp.ones((1, V), jnp.bfloat16)

    def bstep(_, carry):
        lo, hi = carry
        mid = lo + lax.shift_right_logical(hi - lo, 1)
        ind = (dbits <= mid).astype(jnp.bfloat16)
        cnt = lax.dot_general(ones_cnt, ind, (((1,), (0,)), ((), ())),
                              preferred_element_type=jnp.float32)
        pred = cnt >= float(K)
        return jnp.where(pred, lo, mid + 1), jnp.where(pred, mid, hi)

    t, _ = lax.fori_loop(0, 31, bstep,
                         (jnp.zeros((1, BV), jnp.int32),
                          jnp.full((1, BV), 0x7f800000, jnp.int32)))

    ltc = dbits < t
    tiec = dbits == t
    need = (K - jnp.sum(ltc.astype(jnp.int32), axis=0, keepdims=True)
            ).astype(jnp.float32)
    ir = lax.broadcasted_iota(jnp.int32, (V, V), 0)
    ic = lax.broadcasted_iota(jnp.int32, (V, V), 1)
    ltri = (ir > ic).astype(jnp.float32)
    tie_rank = lax.dot_general(ltri, tiec.astype(jnp.float32),
                               (((1,), (0,)), ((), ())),
                               preferred_element_type=jnp.float32)
    selc = ltc | (tiec & (tie_rank < need))
    wmT = jnp.where(selc, jnp.exp(-10.0 * sq), 0.0)

    wmT16 = wmT.astype(jnp.bfloat16)
    lr16 = lr.astype(jnp.bfloat16)
    mean_blocks = []
    max_blocks = []
    for b in range(B):
        wb = wmT[:, b * V:(b + 1) * V]
        lrb = lr[b * V:(b + 1) * V]
        mean_blocks.append(
            lax.dot_general(wb, lrb, (((0,), (0,)), ((), ())),
                            preferred_element_type=jnp.float32,
                            precision=lax.Precision.HIGHEST) * (1.0 / K))
        wb16 = wmT16[:, b * V:(b + 1) * V]
        lrb16 = lr16[b * V:(b + 1) * V]
        rows = [jnp.max(wb16 * lrb16[:, f:f + 1], axis=0, keepdims=True)
                for f in range(N_LR)]
        max_blocks.append(
            jnp.concatenate(rows, axis=0).T.astype(jnp.float32))
    mean_agg = jnp.concatenate(mean_blocks, axis=0)
    max_agg = jnp.concatenate(max_blocks, axis=0)

    fp = jnp.concatenate([x2, mean_agg, max_agg], axis=1)
    out = jnp.dot(fp, woutT_ref[...],
                  preferred_element_type=jnp.float32) + bo_ref[...]
    o_ref[...] = jax.nn.relu(out).reshape(B, V, -1)


def kernel(x, W_slr, b_slr, W_out, b_out):
    B, V_, F = x.shape
    wslrT = W_slr.T
    bslr = b_slr.reshape(1, -1)
    woutT = W_out.T
    bo = b_out.reshape(1, -1)
    n_out = W_out.shape[0]

    full = lambda shape: pl.BlockSpec(shape, lambda: (0,) * len(shape))
    return pl.pallas_call(
        _gravnet_body,
        in_specs=[
            full(x.shape),
            full(wslrT.shape), full(bslr.shape),
            full(woutT.shape), full(bo.shape),
        ],
        out_specs=full((B, V_, n_out)),
        out_shape=jax.ShapeDtypeStruct((B, V_, n_out), jnp.float32),
    )(x, wslrT, bslr, woutT, bo)


# speedup vs baseline: 10.8825x; 1.0870x over previous
import jax
import jax.numpy as jnp
from jax import lax
from jax.experimental import pallas as pl

N_S = 4
N_LR = 22
K = 40
V = 512


def _gravnet_body(x_ref, wslrT_ref, bslr_ref, woutT_ref, bo_ref, o_ref):
    B = x_ref.shape[0]
    x2s = []
    for b in range(B):
        xb = x_ref[b]
        means = jnp.mean(xb, axis=0, keepdims=True)
        x2s.append(jnp.concatenate(
            [xb, jnp.broadcast_to(means, xb.shape)], axis=1))
    x2 = jnp.concatenate(x2s, axis=0)

    slr = jax.nn.relu(jnp.dot(x2, wslrT_ref[...],
                              preferred_element_type=jnp.float32)
                      + bslr_ref[...])
    s = slr[:, :N_S]
    lr = slr[:, N_S:N_S + N_LR]

    ones_row = jnp.ones((1, N_S), jnp.float32)
    sq_blocks = []
    for b in range(B):
        sb = s[b * V:(b + 1) * V]
        u = sb * sb
        ssq_col = jnp.sum(u, axis=1, keepdims=True)
        ssq_row = lax.dot_general(ones_row, u, (((1,), (1,)), ((), ())),
                                  preferred_element_type=jnp.float32,
                                  precision=lax.Precision.HIGHEST)
        gram = lax.dot_general(sb, sb, (((1,), (1,)), ((), ())),
                               preferred_element_type=jnp.float32,
                               precision=lax.Precision.HIGHEST)
        sq_blocks.append(jnp.maximum(ssq_col + ssq_row - 2.0 * gram, 0.0))
    sq = jnp.concatenate(sq_blocks, axis=1)

    dbits = lax.bitcast_convert_type(sq, jnp.int32)
    BV = B * V

    ones_cnt = jnp.ones((1, V), jnp.bfloat16)

    def bstep(_, carry):
        lo, hi = carry
        mid = lo + lax.shift_right_logical(hi - lo, 1)
        ind = (dbits <= mid).astype(jnp.bfloat16)
        cnt = lax.dot_general(ones_cnt, ind, (((1,), (0,)), ((), ())),
                              preferred_element_type=jnp.float32)
        pred = cnt >= float(K)
        return jnp.where(pred, lo, mid + 1), jnp.where(pred, mid, hi)

    t, _ = lax.fori_loop(0, 31, bstep,
                         (jnp.zeros((1, BV), jnp.int32),
                          jnp.full((1, BV), 0x7f800000, jnp.int32)),
                         unroll=4)

    ltc = dbits < t
    tiec = dbits == t
    need = (K - jnp.sum(ltc.astype(jnp.int32), axis=0, keepdims=True)
            ).astype(jnp.float32)
    ir = lax.broadcasted_iota(jnp.int32, (V, V), 0)
    ic = lax.broadcasted_iota(jnp.int32, (V, V), 1)
    ltri = (ir > ic).astype(jnp.float32)
    tie_rank = lax.dot_general(ltri, tiec.astype(jnp.float32),
                               (((1,), (0,)), ((), ())),
                               preferred_element_type=jnp.float32)
    selc = ltc | (tiec & (tie_rank < need))
    wmT = jnp.where(selc, jnp.exp(-10.0 * sq), 0.0)

    wmT16 = wmT.astype(jnp.bfloat16)
    lr16 = lr.astype(jnp.bfloat16)
    mean_blocks = []
    max_blocks = []
    for b in range(B):
        wb = wmT[:, b * V:(b + 1) * V]
        lrb = lr[b * V:(b + 1) * V]
        mean_blocks.append(
            lax.dot_general(wb, lrb, (((0,), (0,)), ((), ())),
                            preferred_element_type=jnp.float32,
                            precision=lax.Precision.HIGHEST) * (1.0 / K))
        wb16 = wmT16[:, b * V:(b + 1) * V]
        lrb16 = lr16[b * V:(b + 1) * V]
        rows = [jnp.max(wb16 * lrb16[:, f:f + 1], axis=0, keepdims=True)
                for f in range(N_LR)]
        max_blocks.append(
            jnp.concatenate(rows, axis=0).T.astype(jnp.float32))
    mean_agg = jnp.concatenate(mean_blocks, axis=0)
    max_agg = jnp.concatenate(max_blocks, axis=0)

    fp = jnp.concatenate([x2, mean_agg, max_agg], axis=1)
    out = jnp.dot(fp, woutT_ref[...],
                  preferred_element_type=jnp.float32) + bo_ref[...]
    o_ref[...] = jax.nn.relu(out).reshape(B, V, -1)


def kernel(x, W_slr, b_slr, W_out, b_out):
    B, V_, F = x.shape
    wslrT = W_slr.T
    bslr = b_slr.reshape(1, -1)
    woutT = W_out.T
    bo = b_out.reshape(1, -1)
    n_out = W_out.shape[0]

    full = lambda shape: pl.BlockSpec(shape, lambda: (0,) * len(shape))
    return pl.pallas_call(
        _gravnet_body,
        in_specs=[
            full(x.shape),
            full(wslrT.shape), full(bslr.shape),
            full(woutT.shape), full(bo.shape),
        ],
        out_specs=full((B, V_, n_out)),
        out_shape=jax.ShapeDtypeStruct((B, V_, n_out), jnp.float32),
    )(x, wslrT, bslr, woutT, bo)


# speedup vs baseline: 11.4039x; 1.0479x over previous
import jax
import jax.numpy as jnp
from jax import lax
from jax.experimental import pallas as pl

N_S = 4
N_LR = 22
K = 40
V = 512


def _gravnet_body(x_ref, wslrT_ref, bslr_ref, woutT_ref, bo_ref, o_ref):
    B = x_ref.shape[0]
    x2s = []
    for b in range(B):
        xb = x_ref[b]
        means = jnp.mean(xb, axis=0, keepdims=True)
        x2s.append(jnp.concatenate(
            [xb, jnp.broadcast_to(means, xb.shape)], axis=1))
    x2 = jnp.concatenate(x2s, axis=0)

    slr = jax.nn.relu(jnp.dot(x2, wslrT_ref[...],
                              preferred_element_type=jnp.float32)
                      + bslr_ref[...])
    s = slr[:, :N_S]
    lr = slr[:, N_S:N_S + N_LR]

    ones_row = jnp.ones((1, N_S), jnp.float32)
    sq_blocks = []
    for b in range(B):
        sb = s[b * V:(b + 1) * V]
        u = sb * sb
        ssq_col = jnp.sum(u, axis=1, keepdims=True)
        ssq_row = lax.dot_general(ones_row, u, (((1,), (1,)), ((), ())),
                                  preferred_element_type=jnp.float32,
                                  precision=lax.Precision.HIGHEST)
        gram = lax.dot_general(sb, sb, (((1,), (1,)), ((), ())),
                               preferred_element_type=jnp.float32,
                               precision=lax.Precision.HIGHEST)
        sq_blocks.append(jnp.maximum(ssq_col + ssq_row - 2.0 * gram, 0.0))
    sq = jnp.concatenate(sq_blocks, axis=1)

    dbits = lax.bitcast_convert_type(sq, jnp.int32)
    BV = B * V

    ones_cnt = jnp.ones((1, V), jnp.bfloat16)

    def bstep(_, carry):
        lo, hi = carry
        mid = lo + lax.shift_right_logical(hi - lo, 1)
        ind = (dbits <= mid).astype(jnp.bfloat16)
        cnt = lax.dot_general(ones_cnt, ind, (((1,), (0,)), ((), ())),
                              preferred_element_type=jnp.float32)
        pred = cnt >= float(K)
        return jnp.where(pred, lo, mid + 1), jnp.where(pred, mid, hi)

    t, _ = lax.fori_loop(0, 31, bstep,
                         (jnp.zeros((1, BV), jnp.int32),
                          jnp.full((1, BV), 0x7f800000, jnp.int32)),
                         unroll=8)

    ltc = dbits < t
    tiec = dbits == t
    need = (K - jnp.sum(ltc.astype(jnp.int32), axis=0, keepdims=True)
            ).astype(jnp.float32)
    ir = lax.broadcasted_iota(jnp.int32, (V, V), 0)
    ic = lax.broadcasted_iota(jnp.int32, (V, V), 1)
    ltri = (ir > ic).astype(jnp.float32)
    tie_rank = lax.dot_general(ltri, tiec.astype(jnp.float32),
                               (((1,), (0,)), ((), ())),
                               preferred_element_type=jnp.float32)
    selc = ltc | (tiec & (tie_rank < need))
    wmT = jnp.where(selc, jnp.exp(-10.0 * sq), 0.0)

    wmT16 = wmT.astype(jnp.bfloat16)
    wmTlo = (wmT - wmT16.astype(jnp.float32)).astype(jnp.bfloat16)
    lr16 = lr.astype(jnp.bfloat16)
    lrlo = (lr - lr16.astype(jnp.float32)).astype(jnp.bfloat16)
    mean_blocks = []
    max_blocks = []
    dn = (((0,), (0,)), ((), ()))
    for b in range(B):
        sl = slice(b * V, (b + 1) * V)
        wb16, wblo = wmT16[:, sl], wmTlo[:, sl]
        lrb16, lrblo = lr16[sl], lrlo[sl]
        acc = (lax.dot_general(wb16, lrb16, dn,
                               preferred_element_type=jnp.float32)
               + lax.dot_general(wb16, lrblo, dn,
                                 preferred_element_type=jnp.float32)
               + lax.dot_general(wblo, lrb16, dn,
                                 preferred_element_type=jnp.float32))
        mean_blocks.append(acc * (1.0 / K))
        rows = [jnp.max(wb16 * lrb16[:, f:f + 1], axis=0, keepdims=True)
                for f in range(N_LR)]
        max_blocks.append(
            jnp.concatenate(rows, axis=0).T.astype(jnp.float32))
    mean_agg = jnp.concatenate(mean_blocks, axis=0)
    max_agg = jnp.concatenate(max_blocks, axis=0)

    fp = jnp.concatenate([x2, mean_agg, max_agg], axis=1)
    out = jnp.dot(fp, woutT_ref[...],
                  preferred_element_type=jnp.float32) + bo_ref[...]
    o_ref[...] = jax.nn.relu(out).reshape(B, V, -1)


def kernel(x, W_slr, b_slr, W_out, b_out):
    B, V_, F = x.shape
    wslrT = W_slr.T
    bslr = b_slr.reshape(1, -1)
    woutT = W_out.T
    bo = b_out.reshape(1, -1)
    n_out = W_out.shape[0]

    full = lambda shape: pl.BlockSpec(shape, lambda: (0,) * len(shape))
    return pl.pallas_call(
        _gravnet_body,
        in_specs=[
            full(x.shape),
            full(wslrT.shape), full(bslr.shape),
            full(woutT.shape), full(bo.shape),
        ],
        out_specs=full((B, V_, n_out)),
        out_shape=jax.ShapeDtypeStruct((B, V_, n_out), jnp.float32),
    )(x, wslrT, bslr, woutT, bo)


# speedup vs baseline: 11.4088x; 1.0004x over previous
import jax
import jax.numpy as jnp
from jax import lax
from jax.experimental import pallas as pl

N_S = 4
N_LR = 22
K = 40
V = 512


def _gravnet_body(x_ref, wslrT_ref, bslr_ref, woutT_ref, bo_ref, o_ref):
    B = x_ref.shape[0]
    x2s = []
    for b in range(B):
        xb = x_ref[b]
        means = jnp.mean(xb, axis=0, keepdims=True)
        x2s.append(jnp.concatenate(
            [xb, jnp.broadcast_to(means, xb.shape)], axis=1))
    x2 = jnp.concatenate(x2s, axis=0)

    slr = jax.nn.relu(jnp.dot(x2, wslrT_ref[...],
                              preferred_element_type=jnp.float32)
                      + bslr_ref[...])
    s = slr[:, :N_S]
    lr = slr[:, N_S:N_S + N_LR]

    ones_row = jnp.ones((1, N_S), jnp.float32)
    sq_blocks = []
    for b in range(B):
        sb = s[b * V:(b + 1) * V]
        u = sb * sb
        ssq_col = jnp.sum(u, axis=1, keepdims=True)
        ssq_row = lax.dot_general(ones_row, u, (((1,), (1,)), ((), ())),
                                  preferred_element_type=jnp.float32,
                                  precision=lax.Precision.HIGHEST)
        gram = lax.dot_general(sb, sb, (((1,), (1,)), ((), ())),
                               preferred_element_type=jnp.float32,
                               precision=lax.Precision.HIGHEST)
        sq_blocks.append(jnp.maximum(ssq_col + ssq_row - 2.0 * gram, 0.0))
    sq = jnp.concatenate(sq_blocks, axis=1)

    dbits = lax.bitcast_convert_type(sq, jnp.int32)
    BV = B * V

    ones_cnt = jnp.ones((1, V), jnp.bfloat16)

    def bstep(_, carry):
        lo, hi = carry
        mid = lo + lax.shift_right_logical(hi - lo, 1)
        ind = (dbits <= mid).astype(jnp.bfloat16)
        cnt = lax.dot_general(ones_cnt, ind, (((1,), (0,)), ((), ())),
                              preferred_element_type=jnp.float32)
        pred = cnt >= float(K)
        return jnp.where(pred, lo, mid + 1), jnp.where(pred, mid, hi)

    t, _ = lax.fori_loop(0, 31, bstep,
                         (jnp.zeros((1, BV), jnp.int32),
                          jnp.full((1, BV), 0x7f800000, jnp.int32)),
                         unroll=8)

    ltc = dbits < t
    tiec = dbits == t
    cnt_lt = lax.dot_general(ones_cnt, ltc.astype(jnp.bfloat16),
                             (((1,), (0,)), ((), ())),
                             preferred_element_type=jnp.float32)
    need = float(K) - cnt_lt
    ir = lax.broadcasted_iota(jnp.int32, (V, V), 0)
    ic = lax.broadcasted_iota(jnp.int32, (V, V), 1)
    ltri = (ir > ic).astype(jnp.float32)
    tie_rank = lax.dot_general(ltri, tiec.astype(jnp.float32),
                               (((1,), (0,)), ((), ())),
                               preferred_element_type=jnp.float32)
    selc = ltc | (tiec & (tie_rank < need))
    wmT = jnp.where(selc, jnp.exp(-10.0 * sq), 0.0)

    wmT16 = wmT.astype(jnp.bfloat16)
    wmTlo = (wmT - wmT16.astype(jnp.float32)).astype(jnp.bfloat16)
    lr16 = lr.astype(jnp.bfloat16)
    lrlo = (lr - lr16.astype(jnp.float32)).astype(jnp.bfloat16)
    mean_blocks = []
    max_blocks = []
    dn = (((0,), (0,)), ((), ()))
    for b in range(B):
        sl = slice(b * V, (b + 1) * V)
        wb16, wblo = wmT16[:, sl], wmTlo[:, sl]
        lrb16, lrblo = lr16[sl], lrlo[sl]
        acc = (lax.dot_general(wb16, lrb16, dn,
                               preferred_element_type=jnp.float32)
               + lax.dot_general(wb16, lrblo, dn,
                                 preferred_element_type=jnp.float32)
               + lax.dot_general(wblo, lrb16, dn,
                                 preferred_element_type=jnp.float32))
        mean_blocks.append(acc * (1.0 / K))
        rows = [jnp.max(wb16 * lrb16[:, f:f + 1], axis=0, keepdims=True)
                for f in range(N_LR)]
        max_blocks.append(
            jnp.concatenate(rows, axis=0).T.astype(jnp.float32))
    mean_agg = jnp.concatenate(mean_blocks, axis=0)
    max_agg = jnp.concatenate(max_blocks, axis=0)

    fp = jnp.concatenate([x2, mean_agg, max_agg], axis=1)
    out = jnp.dot(fp, woutT_ref[...],
                  preferred_element_type=jnp.float32) + bo_ref[...]
    o_ref[...] = jax.nn.relu(out).reshape(B, V, -1)


def kernel(x, W_slr, b_slr, W_out, b_out):
    B, V_, F = x.shape
    wslrT = W_slr.T
    bslr = b_slr.reshape(1, -1)
    woutT = W_out.T
    bo = b_out.reshape(1, -1)
    n_out = W_out.shape[0]

    full = lambda shape: pl.BlockSpec(shape, lambda: (0,) * len(shape))
    return pl.pallas_call(
        _gravnet_body,
        in_specs=[
            full(x.shape),
            full(wslrT.shape), full(bslr.shape),
            full(woutT.shape), full(bo.shape),
        ],
        out_specs=full((B, V_, n_out)),
        out_shape=jax.ShapeDtypeStruct((B, V_, n_out), jnp.float32),
    )(x, wslrT, bslr, woutT, bo)


# speedup vs baseline: 11.7850x; 1.0330x over previous
import jax
import jax.numpy as jnp
from jax import lax
from jax.experimental import pallas as pl

N_S = 4
N_LR = 22
K = 40
V = 512


def _gravnet_body(x_ref, wslrT_ref, bslr_ref, woutT_ref, bo_ref, o_ref):
    B = x_ref.shape[0]
    x2s = []
    for b in range(B):
        xb = x_ref[b]
        means = jnp.mean(xb, axis=0, keepdims=True)
        x2s.append(jnp.concatenate(
            [xb, jnp.broadcast_to(means, xb.shape)], axis=1))
    x2 = jnp.concatenate(x2s, axis=0)

    slr = jax.nn.relu(jnp.dot(x2, wslrT_ref[...],
                              preferred_element_type=jnp.float32)
                      + bslr_ref[...])
    s = slr[:, :N_S]
    lr = slr[:, N_S:N_S + N_LR]

    ones_row = jnp.ones((1, N_S), jnp.float32)
    sq_blocks = []
    for b in range(B):
        sb = s[b * V:(b + 1) * V]
        u = sb * sb
        ssq_col = jnp.sum(u, axis=1, keepdims=True)
        ssq_row = lax.dot_general(ones_row, u, (((1,), (1,)), ((), ())),
                                  preferred_element_type=jnp.float32,
                                  precision=lax.Precision.HIGHEST)
        gram = lax.dot_general(sb, sb, (((1,), (1,)), ((), ())),
                               preferred_element_type=jnp.float32,
                               precision=lax.Precision.HIGHEST)
        sq_blocks.append(jnp.maximum(ssq_col + ssq_row - 2.0 * gram, 0.0))
    sq = jnp.concatenate(sq_blocks, axis=1)

    dbits = lax.bitcast_convert_type(sq, jnp.int32)
    BV = B * V

    ones_cnt = jnp.ones((1, V), jnp.bfloat16)

    def bstep(_, carry):
        lo, hi = carry
        mid = lo + lax.shift_right_logical(hi - lo, 1)
        ind = (dbits <= mid).astype(jnp.bfloat16)
        cnt = lax.dot_general(ones_cnt, ind, (((1,), (0,)), ((), ())),
                              preferred_element_type=jnp.float32)
        pred = cnt >= float(K)
        return jnp.where(pred, lo, mid + 1), jnp.where(pred, mid, hi)

    t, _ = lax.fori_loop(0, 31, bstep,
                         (jnp.zeros((1, BV), jnp.int32),
                          jnp.full((1, BV), 0x7f800000, jnp.int32)),
                         unroll=31)

    ltc = dbits < t
    tiec = dbits == t
    cnt_lt = lax.dot_general(ones_cnt, ltc.astype(jnp.bfloat16),
                             (((1,), (0,)), ((), ())),
                             preferred_element_type=jnp.float32)
    need = float(K) - cnt_lt
    ir = lax.broadcasted_iota(jnp.int32, (V, V), 0)
    ic = lax.broadcasted_iota(jnp.int32, (V, V), 1)
    ltri = (ir > ic).astype(jnp.float32)
    tie_rank = lax.dot_general(ltri, tiec.astype(jnp.float32),
                               (((1,), (0,)), ((), ())),
                               preferred_element_type=jnp.float32)
    selc = ltc | (tiec & (tie_rank < need))
    wmT = jnp.where(selc, jnp.exp(-10.0 * sq), 0.0)

    wmT16 = wmT.astype(jnp.bfloat16)
    wmTlo = (wmT - wmT16.astype(jnp.float32)).astype(jnp.bfloat16)
    lr16 = lr.astype(jnp.bfloat16)
    lrlo = (lr - lr16.astype(jnp.float32)).astype(jnp.bfloat16)
    mean_blocks = []
    max_blocks = []
    dn = (((0,), (0,)), ((), ()))
    for b in range(B):
        sl = slice(b * V, (b + 1) * V)
        wb16, wblo = wmT16[:, sl], wmTlo[:, sl]
        lrb16, lrblo = lr16[sl], lrlo[sl]
        acc = (lax.dot_general(wb16, lrb16, dn,
                               preferred_element_type=jnp.float32)
               + lax.dot_general(wb16, lrblo, dn,
                                 preferred_element_type=jnp.float32)
               + lax.dot_general(wblo, lrb16, dn,
                                 preferred_element_type=jnp.float32))
        mean_blocks.append(acc * (1.0 / K))
        rows = [jnp.max(wb16 * lrb16[:, f:f + 1], axis=0, keepdims=True)
                for f in range(N_LR)]
        max_blocks.append(
            jnp.concatenate(rows, axis=0).T.astype(jnp.float32))
    mean_agg = jnp.concatenate(mean_blocks, axis=0)
    max_agg = jnp.concatenate(max_blocks, axis=0)

    fp = jnp.concatenate([x2, mean_agg, max_agg], axis=1)
    out = jnp.dot(fp, woutT_ref[...],
                  preferred_element_type=jnp.float32) + bo_ref[...]
    o_ref[...] = jax.nn.relu(out).reshape(B, V, -1)


def kernel(x, W_slr, b_slr, W_out, b_out):
    B, V_, F = x.shape
    wslrT = W_slr.T
    bslr = b_slr.reshape(1, -1)
    woutT = W_out.T
    bo = b_out.reshape(1, -1)
    n_out = W_out.shape[0]

    full = lambda shape: pl.BlockSpec(shape, lambda: (0,) * len(shape))
    return pl.pallas_call(
        _gravnet_body,
        in_specs=[
            full(x.shape),
            full(wslrT.shape), full(bslr.shape),
            full(woutT.shape), full(bo.shape),
        ],
        out_specs=full((B, V_, n_out)),
        out_shape=jax.ShapeDtypeStruct((B, V_, n_out), jnp.float32),
    )(x, wslrT, bslr, woutT, bo)


# speedup vs baseline: 11.7987x; 1.0012x over previous
import jax
import jax.numpy as jnp
from jax import lax
from jax.experimental import pallas as pl

N_S = 4
N_LR = 22
K = 40
V = 512


def _gravnet_body(x_ref, wslrT_ref, bslr_ref, woutT_ref, bo_ref, o_ref):
    B = x_ref.shape[0]
    x2s = []
    for b in range(B):
        xb = x_ref[b]
        means = jnp.mean(xb, axis=0, keepdims=True)
        x2s.append(jnp.concatenate(
            [xb, jnp.broadcast_to(means, xb.shape)], axis=1))
    x2 = jnp.concatenate(x2s, axis=0)

    slr = jax.nn.relu(jnp.dot(x2, wslrT_ref[...],
                              preferred_element_type=jnp.float32)
                      + bslr_ref[...])
    s = slr[:, :N_S]
    lr = slr[:, N_S:N_S + N_LR]

    ones_row = jnp.ones((1, N_S), jnp.float32)
    sq_blocks = []
    for b in range(B):
        sb = s[b * V:(b + 1) * V]
        u = sb * sb
        ssq_col = jnp.sum(u, axis=1, keepdims=True)
        ssq_row = lax.dot_general(ones_row, u, (((1,), (1,)), ((), ())),
                                  preferred_element_type=jnp.float32,
                                  precision=lax.Precision.HIGHEST)
        gram = lax.dot_general(sb, sb, (((1,), (1,)), ((), ())),
                               preferred_element_type=jnp.float32,
                               precision=lax.Precision.HIGHEST)
        sq_blocks.append(jnp.maximum(ssq_col + ssq_row - 2.0 * gram, 0.0))
    sq = jnp.concatenate(sq_blocks, axis=1)

    dbits = lax.bitcast_convert_type(sq, jnp.int32)
    BV = B * V

    ones_cnt = jnp.ones((1, V), jnp.bfloat16)
    H = BV // 2
    dbits_h = (dbits[:, :H], dbits[:, H:])

    def bstep(_, carry):
        out = []
        for half, (lo, hi) in zip(dbits_h, carry):
            mid = lo + lax.shift_right_logical(hi - lo, 1)
            ind = (half <= mid).astype(jnp.bfloat16)
            cnt = lax.dot_general(ones_cnt, ind, (((1,), (0,)), ((), ())),
                                  preferred_element_type=jnp.float32)
            pred = cnt >= float(K)
            out.append((jnp.where(pred, lo, mid + 1),
                        jnp.where(pred, mid, hi)))
        return tuple(out)

    init_h = (jnp.zeros((1, H), jnp.int32),
              jnp.full((1, H), 0x7f800000, jnp.int32))
    (t0, _), (t1, _) = lax.fori_loop(0, 31, bstep, (init_h, init_h),
                                     unroll=31)
    t = jnp.concatenate([t0, t1], axis=1)

    ltc = dbits < t
    tiec = dbits == t
    cnt_lt = lax.dot_general(ones_cnt, ltc.astype(jnp.bfloat16),
                             (((1,), (0,)), ((), ())),
                             preferred_element_type=jnp.float32)
    need = float(K) - cnt_lt
    ir = lax.broadcasted_iota(jnp.int32, (V, V), 0)
    ic = lax.broadcasted_iota(jnp.int32, (V, V), 1)
    ltri = (ir > ic).astype(jnp.float32)
    tie_rank = lax.dot_general(ltri, tiec.astype(jnp.float32),
                               (((1,), (0,)), ((), ())),
                               preferred_element_type=jnp.float32)
    selc = ltc | (tiec & (tie_rank < need))
    wmT = jnp.where(selc, jnp.exp(-10.0 * sq), 0.0)

    wmT16 = wmT.astype(jnp.bfloat16)
    wmTlo = (wmT - wmT16.astype(jnp.float32)).astype(jnp.bfloat16)
    lr16 = lr.astype(jnp.bfloat16)
    lrlo = (lr - lr16.astype(jnp.float32)).astype(jnp.bfloat16)
    mean_blocks = []
    max_blocks = []
    dn = (((0,), (0,)), ((), ()))
    for b in range(B):
        sl = slice(b * V, (b + 1) * V)
        wb16, wblo = wmT16[:, sl], wmTlo[:, sl]
        lrb16, lrblo = lr16[sl], lrlo[sl]
        acc = (lax.dot_general(wb16, lrb16, dn,
                               preferred_element_type=jnp.float32)
               + lax.dot_general(wb16, lrblo, dn,
                                 preferred_element_type=jnp.float32)
               + lax.dot_general(wblo, lrb16, dn,
                                 preferred_element_type=jnp.float32))
        mean_blocks.append(acc * (1.0 / K))
        rows = [jnp.max(wb16 * lrb16[:, f:f + 1], axis=0, keepdims=True)
                for f in range(N_LR)]
        max_blocks.append(
            jnp.concatenate(rows, axis=0).T.astype(jnp.float32))
    mean_agg = jnp.concatenate(mean_blocks, axis=0)
    max_agg = jnp.concatenate(max_blocks, axis=0)

    fp = jnp.concatenate([x2, mean_agg, max_agg], axis=1)
    out = jnp.dot(fp, woutT_ref[...],
                  preferred_element_type=jnp.float32) + bo_ref[...]
    o_ref[...] = jax.nn.relu(out).reshape(B, V, -1)


def kernel(x, W_slr, b_slr, W_out, b_out):
    B, V_, F = x.shape
    wslrT = W_slr.T
    bslr = b_slr.reshape(1, -1)
    woutT = W_out.T
    bo = b_out.reshape(1, -1)
    n_out = W_out.shape[0]

    full = lambda shape: pl.BlockSpec(shape, lambda: (0,) * len(shape))
    return pl.pallas_call(
        _gravnet_body,
        in_specs=[
            full(x.shape),
            full(wslrT.shape), full(bslr.shape),
            full(woutT.shape), full(bo.shape),
        ],
        out_specs=full((B, V_, n_out)),
        out_shape=jax.ShapeDtypeStruct((B, V_, n_out), jnp.float32),
    )(x, wslrT, bslr, woutT, bo)
